# baseline (speedup 1.0000x reference)
import functools

import jax
import jax.numpy as jnp
from jax import lax
from jax.experimental import pallas as pl
from jax.experimental.pallas import tpu as pltpu
from jax.experimental.pallas import tpu_sc as plsc

T = 2048
H = 1024
I = 512
E = 8
K = 2
P = T * K
BR = 256
NT = P // BR + E
NPAD = NT * BR
NW = 32
RPW = NPAD // NW
TPW = T // NW


def _router_body(l_ref, o_ref):
    logits = l_ref[...]
    col = lax.broadcasted_iota(jnp.int32, (BR, E), 1)
    m1 = jnp.max(logits, axis=1, keepdims=True)
    a1 = jnp.min(jnp.where(logits >= m1, col, 127), axis=1, keepdims=True)
    l2 = jnp.where(col == a1, -1e30, logits)
    m2 = jnp.max(l2, axis=1, keepdims=True)
    a2 = jnp.min(jnp.where(l2 >= m2, col, 127), axis=1, keepdims=True)
    s = jnp.exp(m2 - m1)
    wa = 1.0 / (1.0 + s)
    wb = s / (1.0 + s)
    o_ref[:, 0:1] = a1.astype(jnp.float32)
    o_ref[:, 1:2] = a2.astype(jnp.float32)
    o_ref[:, 2:3] = wa
    o_ref[:, 3:4] = wb


def _router(logits):
    return pl.pallas_call(
        _router_body,
        grid=(T // BR,),
        in_specs=[
            pl.BlockSpec((BR, E), lambda i: (i, 0)),
        ],
        out_specs=pl.BlockSpec((BR, 4), lambda i: (i, 0)),
        out_shape=jax.ShapeDtypeStruct((T, 4), jnp.float32),
    )(logits)


def _dispatch_body(eidx_hbm, wflat_hbm, hidden_hbm, xs_hbm, ws_hbm, pos_hbm,
                   meta_hbm, eidx_v, wfl_v, rm_v, pos_v, meta_v, idx_v, rows_v,
                   wbuf_v, sem):
    wid = lax.axis_index("c") * 16 + lax.axis_index("s")

    pltpu.sync_copy(eidx_hbm, eidx_v)
    pltpu.sync_copy(wflat_hbm, wfl_v)

    def zbody(i, c):
        rm_v[pl.ds(i * 16, 16)] = jnp.zeros((16,), jnp.int32)
        return c
    lax.fori_loop(0, NPAD // 16, zbody, 0)

    def cbody(i, cs):
        ch = eidx_v[pl.ds(i * 16, 16)]
        return tuple(cs[e] + plsc.all_reduce_population_count(ch == e)
                     for e in range(E))
    counts = lax.fori_loop(0, P // 16, cbody,
                           (jnp.zeros((16,), jnp.int32),) * E)

    offs = []
    o = jnp.zeros((16,), jnp.int32)
    for e in range(E):
        offs.append(o)
        pad = lax.shift_left(
            lax.shift_right_logical(counts[e] + (BR - 1), 8), 8)
        o = o + pad

    ones16 = jnp.full((16,), 1, jnp.int32)
    def sbody(i, carry):
        pid = carry[0]
        wps = carry[1:]
        ch = eidx_v[pl.ds(i * 16, 16)]
        new = [pid + 16]
        for e in range(E):
            m = ch == e
            mi = m.astype(jnp.int32)
            rp = plsc.cumsum(mi) + wps[e] - ones16
            plsc.store_scatter(rm_v, [rp], pid, mask=m)
            plsc.store_scatter(pos_v, [pid], rp, mask=m)
            new.append(wps[e] + plsc.all_reduce_population_count(m))
        return tuple(new)
    lax.fori_loop(0, P // 16, sbody,
                  (lax.iota(jnp.int32, 16),) + tuple(offs))

    @pl.when(wid == 0)
    def _():
        for ch in range(NW // 16):
            rowv = (lax.iota(jnp.int32, 16) + ch * 16) * BR
            ex = jnp.zeros((16,), jnp.int32)
            for e in range(1, E):
                ex = ex + (offs[e] <= rowv).astype(jnp.int32)
            oe = jnp.zeros((16,), jnp.int32)
            ce = jnp.zeros((16,), jnp.int32)
            for e in range(E):
                sel = ex == e
                oe = jnp.where(sel, offs[e], oe)
                ce = jnp.where(sel, counts[e], ce)
            meta_v[0, pl.ds(ch * 16, 16)] = ex
            meta_v[1, pl.ds(ch * 16, 16)] = (rowv - oe < ce).astype(jnp.int32)
        pltpu.sync_copy(meta_v, meta_hbm)
        pltpu.sync_copy(pos_v, pos_hbm)

    base = wid * RPW
    def gbody(c, carry):
        rb = base + c * 32
        for h in range(2):
            rm16 = rm_v[pl.ds(rb + h * 16, 16)]
            idx_v[pl.ds(h * 16, 16)] = lax.shift_right_logical(rm16, 1)
            wbuf_v[pl.ds(h * 16, 16)] = plsc.load_gather(wfl_v, [rm16])
        pltpu.async_copy(hidden_hbm.at[idx_v], rows_v, sem).wait()
        pltpu.sync_copy(rows_v, xs_hbm.at[pl.ds(rb, 32)])
        pltpu.sync_copy(wbuf_v, ws_hbm.at[pl.ds(rb, 32)])
        return carry
    lax.fori_loop(0, RPW // 32, gbody, 0)


def _dispatch(eidx, wflat, hidden):
    mesh = plsc.VectorSubcoreMesh(core_axis_name="c", subcore_axis_name="s")
    fn = functools.partial(
        pl.kernel,
        mesh=mesh,
        out_type=[
            jax.ShapeDtypeStruct((NPAD, H), jnp.float32),
            jax.ShapeDtypeStruct((NPAD,), jnp.float32),
            jax.ShapeDtypeStruct((P,), jnp.int32),
            jax.ShapeDtypeStruct((2, NW), jnp.int32),
        ],
        scratch_types=[
            pltpu.VMEM((P,), jnp.int32),
            pltpu.VMEM((P,), jnp.float32),
            pltpu.VMEM((NPAD,), jnp.int32),
            pltpu.VMEM((P,), jnp.int32),
            pltpu.VMEM((2, NW), jnp.int32),
            pltpu.VMEM((32,), jnp.int32),
            pltpu.VMEM((32, H), jnp.float32),
            pltpu.VMEM((32,), jnp.float32),
            pltpu.SemaphoreType.DMA,
        ],
        compiler_params=pltpu.CompilerParams(needs_layout_passes=False),
    )(_dispatch_body)
    return fn(eidx, wflat, hidden)


def _ffn_body(meta_ref, x_ref, w1_ref, w3_ref, w2_ref, ws_ref, o_ref):
    i = pl.program_id(0)

    @pl.when(meta_ref[1, i] == 1)
    def _():
        x = x_ref[...]
        h1 = lax.dot_general(x, w1_ref[0], (((1,), (1,)), ((), ())),
                             precision=lax.Precision.HIGHEST,
                             preferred_element_type=jnp.float32)
        h3 = lax.dot_general(x, w3_ref[0], (((1,), (1,)), ((), ())),
                             precision=lax.Precision.HIGHEST,
                             preferred_element_type=jnp.float32)
        act = h1 * lax.logistic(h1) * h3
        out = lax.dot_general(act, w2_ref[0], (((1,), (1,)), ((), ())),
                              precision=lax.Precision.HIGHEST,
                              preferred_element_type=jnp.float32)
        o_ref[...] = out * ws_ref[...]


def _ffn(meta, xs, ws, w1, w3, w2):
    grid_spec = pltpu.PrefetchScalarGridSpec(
        num_scalar_prefetch=1,
        grid=(NT,),
        in_specs=[
            pl.BlockSpec((BR, H), lambda i, m: (i, 0)),
            pl.BlockSpec((1, I, H), lambda i, m: (m[0, i], 0, 0)),
            pl.BlockSpec((1, I, H), lambda i, m: (m[0, i], 0, 0)),
            pl.BlockSpec((1, H, I), lambda i, m: (m[0, i], 0, 0)),
            pl.BlockSpec((BR, 1), lambda i, m: (i, 0)),
        ],
        out_specs=pl.BlockSpec((BR, H), lambda i, m: (i, 0)),
    )
    return pl.pallas_call(
        _ffn_body,
        grid_spec=grid_spec,
        out_shape=jax.ShapeDtypeStruct((NPAD, H), jnp.float32),
    )(meta, xs, w1, w3, w2, ws.reshape(NPAD, 1))


def _combine_body(ys_hbm, pos_hbm, out_hbm, pos_v, idx_v, rows_v, acc_v, sem):
    wid = lax.axis_index("c") * 16 + lax.axis_index("s")
    tb = wid * TPW
    pltpu.sync_copy(pos_hbm.at[pl.ds(tb * K, TPW * K)], pos_v)

    for c in range(TPW // 16):
        for h in range(2):
            idx_v[pl.ds(h * 16, 16)] = pos_v[pl.ds(c * 32 + h * 16, 16)]
        pltpu.async_copy(ys_hbm.at[idx_v], rows_v, sem).wait()

        def qbody(q, cc):
            for j in range(16):
                a = rows_v[2 * j, pl.ds(q * 16, 16)]
                b = rows_v[2 * j + 1, pl.ds(q * 16, 16)]
                acc_v[j, pl.ds(q * 16, 16)] = a + b
            return cc
        lax.fori_loop(0, H // 16, qbody, 0)
        pltpu.sync_copy(acc_v, out_hbm.at[pl.ds(tb + c * 16, 16)])


def _combine(ys, pos):
    mesh = plsc.VectorSubcoreMesh(core_axis_name="c", subcore_axis_name="s")
    fn = functools.partial(
        pl.kernel,
        mesh=mesh,
        out_type=jax.ShapeDtypeStruct((T, H), jnp.float32),
        scratch_types=[
            pltpu.VMEM((TPW * K,), jnp.int32),
            pltpu.VMEM((32,), jnp.int32),
            pltpu.VMEM((32, H), jnp.float32),
            pltpu.VMEM((16, H), jnp.float32),
            pltpu.SemaphoreType.DMA,
        ],
        compiler_params=pltpu.CompilerParams(needs_layout_passes=False),
    )(_combine_body)
    return fn(ys, pos)


def kernel(hidden_states, gate_w, w1, w3, w2):
    logits = jnp.dot(hidden_states.astype(jnp.float32),
                     gate_w.astype(jnp.float32).T)
    r = _router(logits)
    eidx = r[:, 0:2].astype(jnp.int32).reshape(-1)
    wflat = r[:, 2:4].reshape(-1)
    xs, ws, pos, meta = _dispatch(eidx, wflat, hidden_states)
    ys = _ffn(meta, xs, ws, w1, w3, w2)
    return _combine(ys, pos)


# speedup vs baseline: 1.2662x; 1.2662x over previous
import functools

import jax
import jax.numpy as jnp
from jax import lax
from jax.experimental import pallas as pl
from jax.experimental.pallas import tpu as pltpu
from jax.experimental.pallas import tpu_sc as plsc

T = 2048
H = 1024
I = 512
E = 8
K = 2
P = T * K
BR = 256
NT = P // BR + E
NPAD = NT * BR
NW = 32
RPW = NPAD // NW
TPW = T // NW


def _router_body(l_ref, o_ref):
    logits = l_ref[...]
    col = lax.broadcasted_iota(jnp.int32, (BR, E), 1)
    m1 = jnp.max(logits, axis=1, keepdims=True)
    a1 = jnp.min(jnp.where(logits >= m1, col, 127), axis=1, keepdims=True)
    l2 = jnp.where(col == a1, -1e30, logits)
    m2 = jnp.max(l2, axis=1, keepdims=True)
    a2 = jnp.min(jnp.where(l2 >= m2, col, 127), axis=1, keepdims=True)
    s = jnp.exp(m2 - m1)
    wa = 1.0 / (1.0 + s)
    wb = s / (1.0 + s)
    o_ref[:, 0:1] = a1.astype(jnp.float32)
    o_ref[:, 1:2] = a2.astype(jnp.float32)
    o_ref[:, 2:3] = wa
    o_ref[:, 3:4] = wb


def _router(logits):
    return pl.pallas_call(
        _router_body,
        grid=(T // BR,),
        in_specs=[
            pl.BlockSpec((BR, E), lambda i: (i, 0)),
        ],
        out_specs=pl.BlockSpec((BR, 4), lambda i: (i, 0)),
        out_shape=jax.ShapeDtypeStruct((T, 4), jnp.float32),
    )(logits)


def _dispatch_body(eidx_hbm, wflat_hbm, hidden_hbm, xs_hbm, ws_hbm, pos_hbm,
                   meta_hbm, eidx_v, wfl_v, rm_v, pos_v, meta_v, idx_v, rows_v,
                   wbuf_v, sem):
    wid = lax.axis_index("c") * 16 + lax.axis_index("s")

    pltpu.sync_copy(eidx_hbm, eidx_v)
    pltpu.sync_copy(wflat_hbm, wfl_v)

    def zbody(i, c):
        rm_v[pl.ds(i * 16, 16)] = jnp.zeros((16,), jnp.int32)
        return c
    lax.fori_loop(0, NPAD // 16, zbody, 0)

    def cbody(i, cs):
        ch = eidx_v[pl.ds(i * 16, 16)]
        return tuple(cs[e] + plsc.all_reduce_population_count(ch == e)
                     for e in range(E))
    counts = lax.fori_loop(0, P // 16, cbody,
                           (jnp.zeros((16,), jnp.int32),) * E)

    offs = []
    o = jnp.zeros((16,), jnp.int32)
    for e in range(E):
        offs.append(o)
        pad = lax.shift_left(
            lax.shift_right_logical(counts[e] + (BR - 1), 8), 8)
        o = o + pad

    ones16 = jnp.full((16,), 1, jnp.int32)
    def sbody(i, carry):
        pid = carry[0]
        wps = carry[1:]
        ch = eidx_v[pl.ds(i * 16, 16)]
        new = [pid + 16]
        for e in range(E):
            m = ch == e
            mi = m.astype(jnp.int32)
            rp = plsc.cumsum(mi) + wps[e] - ones16
            plsc.store_scatter(rm_v, [rp], pid, mask=m)
            plsc.store_scatter(pos_v, [pid], rp, mask=m)
            new.append(wps[e] + plsc.all_reduce_population_count(m))
        return tuple(new)
    lax.fori_loop(0, P // 16, sbody,
                  (lax.iota(jnp.int32, 16),) + tuple(offs))

    @pl.when(wid == 0)
    def _():
        for ch in range(NW // 16):
            rowv = (lax.iota(jnp.int32, 16) + ch * 16) * BR
            ex = jnp.zeros((16,), jnp.int32)
            for e in range(1, E):
                ex = ex + (offs[e] <= rowv).astype(jnp.int32)
            oe = jnp.zeros((16,), jnp.int32)
            ce = jnp.zeros((16,), jnp.int32)
            for e in range(E):
                sel = ex == e
                oe = jnp.where(sel, offs[e], oe)
                ce = jnp.where(sel, counts[e], ce)
            meta_v[0, pl.ds(ch * 16, 16)] = ex
            meta_v[1, pl.ds(ch * 16, 16)] = (rowv - oe < ce).astype(jnp.int32)
        pltpu.sync_copy(meta_v, meta_hbm)
        pltpu.sync_copy(pos_v, pos_hbm)

    base = wid * RPW
    def gbody(c, carry):
        rb = base + c * 32
        for h in range(2):
            rm16 = rm_v[pl.ds(rb + h * 16, 16)]
            idx_v[pl.ds(h * 16, 16)] = lax.shift_right_logical(rm16, 1)
            wbuf_v[pl.ds(h * 16, 16)] = plsc.load_gather(wfl_v, [rm16])
        pltpu.async_copy(hidden_hbm.at[idx_v], rows_v, sem).wait()
        pltpu.sync_copy(rows_v, xs_hbm.at[pl.ds(rb, 32)])
        pltpu.sync_copy(wbuf_v, ws_hbm.at[pl.ds(rb, 32)])
        return carry
    lax.fori_loop(0, RPW // 32, gbody, 0)


def _dispatch(eidx, wflat, hidden):
    mesh = plsc.VectorSubcoreMesh(core_axis_name="c", subcore_axis_name="s")
    fn = functools.partial(
        pl.kernel,
        mesh=mesh,
        out_type=[
            jax.ShapeDtypeStruct((NPAD, H), jnp.float32),
            jax.ShapeDtypeStruct((NPAD,), jnp.float32),
            jax.ShapeDtypeStruct((P,), jnp.int32),
            jax.ShapeDtypeStruct((2, NW), jnp.int32),
        ],
        scratch_types=[
            pltpu.VMEM((P,), jnp.int32),
            pltpu.VMEM((P,), jnp.float32),
            pltpu.VMEM((NPAD,), jnp.int32),
            pltpu.VMEM((P,), jnp.int32),
            pltpu.VMEM((2, NW), jnp.int32),
            pltpu.VMEM((32,), jnp.int32),
            pltpu.VMEM((32, H), jnp.float32),
            pltpu.VMEM((32,), jnp.float32),
            pltpu.SemaphoreType.DMA,
        ],
        compiler_params=pltpu.CompilerParams(needs_layout_passes=False),
    )(_dispatch_body)
    return fn(eidx, wflat, hidden)


def _ffn_body(meta_ref, x_ref, w1_ref, w3_ref, w2_ref, ws_ref, o_ref):
    i = pl.program_id(0)

    @pl.when(meta_ref[1, i] == 1)
    def _():
        x = x_ref[...]
        h1 = lax.dot_general(x, w1_ref[0], (((1,), (1,)), ((), ())),
                             preferred_element_type=jnp.float32)
        h3 = lax.dot_general(x, w3_ref[0], (((1,), (1,)), ((), ())),
                             preferred_element_type=jnp.float32)
        act = h1 * lax.logistic(h1) * h3
        out = lax.dot_general(act, w2_ref[0], (((1,), (1,)), ((), ())),
                              preferred_element_type=jnp.float32)
        o_ref[...] = out * ws_ref[...]


def _ffn(meta, xs, ws, w1, w3, w2):
    grid_spec = pltpu.PrefetchScalarGridSpec(
        num_scalar_prefetch=1,
        grid=(NT,),
        in_specs=[
            pl.BlockSpec((BR, H), lambda i, m: (i, 0)),
            pl.BlockSpec((1, I, H), lambda i, m: (m[0, i], 0, 0)),
            pl.BlockSpec((1, I, H), lambda i, m: (m[0, i], 0, 0)),
            pl.BlockSpec((1, H, I), lambda i, m: (m[0, i], 0, 0)),
            pl.BlockSpec((BR, 1), lambda i, m: (i, 0)),
        ],
        out_specs=pl.BlockSpec((BR, H), lambda i, m: (i, 0)),
    )
    return pl.pallas_call(
        _ffn_body,
        grid_spec=grid_spec,
        out_shape=jax.ShapeDtypeStruct((NPAD, H), jnp.float32),
    )(meta, xs, w1, w3, w2, ws.reshape(NPAD, 1))


def _combine_body(ys_hbm, pos_hbm, out_hbm, pos_v, idx_v, rows_v, acc_v, sem):
    wid = lax.axis_index("c") * 16 + lax.axis_index("s")
    tb = wid * TPW
    pltpu.sync_copy(pos_hbm.at[pl.ds(tb * K, TPW * K)], pos_v)

    for c in range(TPW // 16):
        for h in range(2):
            idx_v[pl.ds(h * 16, 16)] = pos_v[pl.ds(c * 32 + h * 16, 16)]
        pltpu.async_copy(ys_hbm.at[idx_v], rows_v, sem).wait()

        def qbody(q, cc):
            for j in range(16):
                a = rows_v[2 * j, pl.ds(q * 16, 16)]
                b = rows_v[2 * j + 1, pl.ds(q * 16, 16)]
                acc_v[j, pl.ds(q * 16, 16)] = a + b
            return cc
        lax.fori_loop(0, H // 16, qbody, 0)
        pltpu.sync_copy(acc_v, out_hbm.at[pl.ds(tb + c * 16, 16)])


def _combine(ys, pos):
    mesh = plsc.VectorSubcoreMesh(core_axis_name="c", subcore_axis_name="s")
    fn = functools.partial(
        pl.kernel,
        mesh=mesh,
        out_type=jax.ShapeDtypeStruct((T, H), jnp.float32),
        scratch_types=[
            pltpu.VMEM((TPW * K,), jnp.int32),
            pltpu.VMEM((32,), jnp.int32),
            pltpu.VMEM((32, H), jnp.float32),
            pltpu.VMEM((16, H), jnp.float32),
            pltpu.SemaphoreType.DMA,
        ],
        compiler_params=pltpu.CompilerParams(needs_layout_passes=False),
    )(_combine_body)
    return fn(ys, pos)


def kernel(hidden_states, gate_w, w1, w3, w2):
    logits = jnp.dot(hidden_states.astype(jnp.float32),
                     gate_w.astype(jnp.float32).T)
    r = _router(logits)
    eidx = r[:, 0:2].astype(jnp.int32).reshape(-1)
    wflat = r[:, 2:4].reshape(-1)
    xs, ws, pos, meta = _dispatch(eidx, wflat, hidden_states)
    ys = _ffn(meta, xs, ws, w1, w3, w2)
    return _combine(ys, pos)
